# baseline (speedup 1.0000x reference)
import functools

import jax
import jax.numpy as jnp
from jax import lax
from jax.experimental import pallas as pl
from jax.experimental.pallas import tpu as pltpu
from jax.experimental.pallas import tpu_sc as plsc

_NUM_CORES = 2
_NUM_SUBCORES = 16
_NW = _NUM_CORES * _NUM_SUBCORES
_L = 16

_GATHER = 128
_CHUNK = 1024


def _build_transpose_call(v, d):
    assert d == 16
    vpad = v % 128
    full_blocks = v // 128
    assert full_blocks % 4 == 0
    groups = full_blocks // 4
    gcols = 512
    gwords = gcols * d
    iters = (groups + _NW - 1) // _NW

    mesh = plsc.VectorSubcoreMesh(
        core_axis_name="c", subcore_axis_name="s",
        num_cores=_NUM_CORES, num_subcores=_NUM_SUBCORES)

    @functools.partial(
        pl.kernel,
        out_type=[jax.ShapeDtypeStruct((v * d,), jnp.float32),
                  jax.ShapeDtypeStruct((v * d,), jnp.float32)],
        mesh=mesh,
        compiler_params=pltpu.CompilerParams(
            use_tc_tiling_on_sc=True, needs_layout_passes=False),
        scratch_types=[
            [pltpu.VMEM((d, gcols), jnp.float32)] * 4,
            [pltpu.VMEM((gwords,), jnp.float32)] * 4,
            [pltpu.SemaphoreType.DMA] * 4,
        ],
    )
    def tr_call(wt_hbm, et_hbm, wtail_hbm, etail_hbm, wo_hbm, eo_hbm,
                a_bufs, l_bufs, sems):
        wid = lax.axis_index("s") * _NUM_CORES + lax.axis_index("c")
        a_w0, a_e0, a_w1, a_e1 = a_bufs
        l_w0, l_e0, l_w1, l_e1 = l_bufs
        sin0, sout0, sin1, sout1 = sems
        iota = lax.iota(jnp.int32, _L)

        def fire_in(i, a_w, a_e, sin):
            col0 = pl.multiple_of((wid + i * _NW) * gcols, gcols)
            pltpu.async_copy(wt_hbm.at[:, pl.ds(col0, gcols)], a_w, sin)
            pltpu.async_copy(et_hbm.at[:, pl.ds(col0, gcols)], a_e, sin)

        def wait_in(a_w, a_e, sin):
            pltpu.make_async_copy(
                wt_hbm.at[:, pl.ds(0, gcols)], a_w, sin).wait()
            pltpu.make_async_copy(
                et_hbm.at[:, pl.ds(0, gcols)], a_e, sin).wait()

        def fire_out(i, l_w, l_e, sout):
            off = pl.multiple_of((wid + i * _NW) * gwords, gwords)
            pltpu.async_copy(l_w, wo_hbm.at[pl.ds(off, gwords)], sout)
            pltpu.async_copy(l_e, eo_hbm.at[pl.ds(off, gwords)], sout)

        def wait_out(l_w, l_e, sout):
            pltpu.make_async_copy(
                l_w, wo_hbm.at[pl.ds(0, gwords)], sout).wait()
            pltpu.make_async_copy(
                l_e, eo_hbm.at[pl.ds(0, gwords)], sout).wait()

        def do_group(src_v, dst_v):
            @plsc.parallel_loop(0, d, unroll=4)
            def _(dd):
                base_d = iota * d + dd
                for k in range(gcols // _L):
                    val = src_v[dd, pl.ds(k * _L, _L)]
                    plsc.store_scatter(dst_v, [base_d + k * _L * d], val)

        fire_in(0, a_w0, a_e0, sin0)
        fire_in(1, a_w1, a_e1, sin1)

        def sub(p, i, valid_next, a_w, a_e, l_w, l_e, sin, sout):
            g = wid + i * _NW

            @pl.when(g < groups)
            def _():
                wait_in(a_w, a_e, sin)

                @pl.when(p > 0)
                def _():
                    wait_out(l_w, l_e, sout)

                do_group(a_w, l_w)
                do_group(a_e, l_e)
                fire_out(i, l_w, l_e, sout)

                @pl.when(valid_next & (g + 2 * _NW < groups))
                def _():
                    fire_in(i + 2, a_w, a_e, sin)

        def ploop(p, carry):
            sub(p, 2 * p, True, a_w0, a_e0, l_w0, l_e0, sin0, sout0)
            sub(p, 2 * p + 1, True, a_w1, a_e1, l_w1, l_e1, sin1, sout1)
            return carry

        lax.fori_loop(0, (iters + 1) // 2, ploop, 0)
        n0 = (groups - wid + 2 * _NW - 1) // (2 * _NW)
        n1 = (groups - wid - _NW + 2 * _NW - 1) // (2 * _NW)

        @pl.when(n0 > 0)
        def _():
            wait_out(l_w0, l_e0, sout0)

        @pl.when(n1 > 0)
        def _():
            wait_out(l_w1, l_e1, sout1)

        if vpad:
            @pl.when(wid == _NW - 1)
            def _():
                nw = vpad * d
                offt = full_blocks * 128 * d
                pltpu.sync_copy(wtail_hbm, l_w0.at[pl.ds(0, nw)])
                pltpu.sync_copy(etail_hbm, l_e0.at[pl.ds(0, nw)])
                pltpu.sync_copy(l_w0.at[pl.ds(0, nw)],
                                wo_hbm.at[pl.ds(offt, nw)])
                pltpu.sync_copy(l_e0.at[pl.ds(0, nw)],
                                eo_hbm.at[pl.ds(offt, nw)])

    return tr_call


def _build_sc_call(b, f, v, d):
    n = b * f
    assert n % (_NW * _CHUNK) == 0
    per_w = n // (_NW * _CHUNK)
    k = _CHUNK // _GATHER
    blocks_per_f = b // _CHUNK

    mesh = plsc.VectorSubcoreMesh(
        core_axis_name="c", subcore_axis_name="s",
        num_cores=_NUM_CORES, num_subcores=_NUM_SUBCORES)

    @functools.partial(
        pl.kernel,
        out_type=jax.ShapeDtypeStruct((f * d * b,), jnp.float32),
        mesh=mesh,
        compiler_params=pltpu.CompilerParams(
            use_tc_tiling_on_sc=False, needs_layout_passes=False),
        scratch_types=[
            [pltpu.VMEM((k, _GATHER), jnp.int32)] * 2,
            [pltpu.VMEM((_CHUNK, d), jnp.float32)] * 4,
            [pltpu.VMEM((d * _CHUNK,), jnp.float32)] * 2,
            pltpu.VMEM((_L,), jnp.float32),
            [pltpu.SemaphoreType.DMA] * 4,
        ],
    )
    def sc_call(idx_hbm, w_hbm, e_hbm, sig_hbm, out_hbm,
                idx_bufs, ge_bufs, o_bufs, sig_v, sems):
        wid = lax.axis_index("s") * _NUM_CORES + lax.axis_index("c")
        idx0, idx1 = idx_bufs
        w_v0, e_v0, w_v1, e_v1 = ge_bufs
        o_v0, o_v1 = o_bufs
        sin0, sout0, sin1, sout1 = sems
        pltpu.sync_copy(sig_hbm, sig_v)
        sig = sig_v[...]
        iota = lax.iota(jnp.int32, _L)

        def fire_in(i, idx_b, w_b, e_b, sin):
            cg = wid * per_w + i
            pltpu.sync_copy(
                idx_hbm.at[pl.ds(pl.multiple_of(cg * k, 8), k)], idx_b)
            for j in range(k):
                pltpu.async_copy(
                    w_hbm.at[idx_b.at[j]],
                    w_b.at[pl.ds(j * _GATHER, _GATHER)], sin)
                pltpu.async_copy(
                    e_hbm.at[idx_b.at[j]],
                    e_b.at[pl.ds(j * _GATHER, _GATHER)], sin)

        def wait_in(idx_b, w_b, e_b, sin):
            for j in range(k):
                pltpu.make_async_copy(
                    w_hbm.at[idx_b.at[j]],
                    w_b.at[pl.ds(j * _GATHER, _GATHER)], sin).wait()
                pltpu.make_async_copy(
                    e_hbm.at[idx_b.at[j]],
                    e_b.at[pl.ds(j * _GATHER, _GATHER)], sin).wait()

        ltile = (iota // 8) * (8 * _CHUNK) + (iota % 8) * _GATHER
        half = 8 * _CHUNK

        def fire_out(i, o_b, sout):
            cg = wid * per_w + i
            fi = cg // blocks_per_f
            tj0 = (cg % blocks_per_f) * (_CHUNK // _GATHER)
            for t in range(2):
                off = pl.multiple_of(
                    ((fi * 2 + t) * _GATHER + tj0) * (8 * _GATHER), 1024)
                pltpu.async_copy(o_b.at[pl.ds(t * half, half)],
                                 out_hbm.at[pl.ds(off, half)], sout)

        def wait_out(o_b, sout):
            for t in range(2):
                pltpu.make_async_copy(
                    o_b.at[pl.ds(t * half, half)],
                    out_hbm.at[pl.ds(0, half)], sout).wait()

        def compute(w_b, e_b, o_b):
            for jb in range(_CHUNK // _GATHER):
                @plsc.parallel_loop(0, _GATHER, unroll=8)
                def _(j, jb=jb):
                    row = jb * _GATHER + j
                    val = w_b[row] + sig * e_b[row]
                    plsc.store_scatter(o_b, [ltile + (jb * _CHUNK + j)],
                                       val)

        fire_in(0, idx0, w_v0, e_v0, sin0)
        fire_in(1, idx1, w_v1, e_v1, sin1)

        def ploop(p, carry):
            i = 2 * p
            wait_in(idx0, w_v0, e_v0, sin0)

            @pl.when(p > 0)
            def _():
                wait_out(o_v0, sout0)

            compute(w_v0, e_v0, o_v0)
            fire_out(i, o_v0, sout0)
            fire_in(i + 2, idx0, w_v0, e_v0, sin0)

            wait_in(idx1, w_v1, e_v1, sin1)

            @pl.when(p > 0)
            def _():
                wait_out(o_v1, sout1)

            compute(w_v1, e_v1, o_v1)
            fire_out(i + 1, o_v1, sout1)

            @pl.when(p < per_w // 2 - 1)
            def _():
                fire_in(i + 3, idx1, w_v1, e_v1, sin1)
            return carry

        lax.fori_loop(0, per_w // 2, ploop, 0)
        wait_in(idx0, w_v0, e_v0, sin0)
        wait_out(o_v0, sout0)
        compute(w_v0, e_v0, o_v0)
        fire_out(per_w - 1, o_v0, sout0)
        wait_out(o_v0, sout0)
        wait_out(o_v1, sout1)

    return sc_call


def kernel(x, weight, raw_sigma, eps):
    b, f = x.shape
    v, d = weight.shape
    n = b * f
    sigma = jax.nn.softplus(raw_sigma)
    sig_arr = jnp.full((_L,), sigma, dtype=jnp.float32)
    idx = x.T.reshape(n // _GATHER, _GATHER).astype(jnp.int32)
    tr_call = _build_transpose_call(v, d)
    vfull = (v // 128) * 128
    w1d, e1d = tr_call(weight.T, eps.T,
                       weight[vfull:, :].reshape(-1),
                       eps[vfull:, :].reshape(-1))
    w_lin = w1d.reshape(v, d)
    e_lin = e1d.reshape(v, d)
    sc_call = _build_sc_call(b, f, v, d)
    out1d = sc_call(idx, w_lin, e_lin, sig_arr)
    o5 = out1d.reshape(f, d // 8, b // _GATHER, 8, _GATHER)
    return jnp.transpose(o5, (2, 4, 0, 1, 3)).reshape(b, f, d)


# speedup vs baseline: 1.0095x; 1.0095x over previous
import functools

import jax
import jax.numpy as jnp
from jax import lax
from jax.experimental import pallas as pl
from jax.experimental.pallas import tpu as pltpu
from jax.experimental.pallas import tpu_sc as plsc

_NUM_CORES = 2
_NUM_SUBCORES = 16
_NW = _NUM_CORES * _NUM_SUBCORES
_L = 16

_GATHER = 128
_CHUNK = 1024


def _build_transpose_call(v, d):
    assert d == 16
    vpad = v % 128
    full_blocks = v // 128
    assert full_blocks % 4 == 0
    groups = full_blocks // 4
    gcols = 512
    gwords = gcols * d
    iters = (groups + _NW - 1) // _NW

    mesh = plsc.VectorSubcoreMesh(
        core_axis_name="c", subcore_axis_name="s",
        num_cores=_NUM_CORES, num_subcores=_NUM_SUBCORES)

    @functools.partial(
        pl.kernel,
        out_type=[jax.ShapeDtypeStruct((v * d,), jnp.float32),
                  jax.ShapeDtypeStruct((v * d,), jnp.float32)],
        mesh=mesh,
        compiler_params=pltpu.CompilerParams(
            use_tc_tiling_on_sc=True, needs_layout_passes=False),
        scratch_types=[
            [pltpu.VMEM((d, gcols), jnp.float32)] * 4,
            [pltpu.VMEM((gwords,), jnp.float32)] * 4,
            [pltpu.SemaphoreType.DMA] * 4,
        ],
    )
    def tr_call(wt_hbm, et_hbm, wtail_hbm, etail_hbm, wo_hbm, eo_hbm,
                a_bufs, l_bufs, sems):
        wid = lax.axis_index("s") * _NUM_CORES + lax.axis_index("c")
        a_w0, a_e0, a_w1, a_e1 = a_bufs
        l_w0, l_e0, l_w1, l_e1 = l_bufs
        sin0, sout0, sin1, sout1 = sems
        iota = lax.iota(jnp.int32, _L)

        def fire_in(i, a_w, a_e, sin):
            col0 = pl.multiple_of((wid + i * _NW) * gcols, gcols)
            pltpu.async_copy(wt_hbm.at[:, pl.ds(col0, gcols)], a_w, sin)
            pltpu.async_copy(et_hbm.at[:, pl.ds(col0, gcols)], a_e, sin)

        def wait_in(a_w, a_e, sin):
            pltpu.make_async_copy(
                wt_hbm.at[:, pl.ds(0, gcols)], a_w, sin).wait()
            pltpu.make_async_copy(
                et_hbm.at[:, pl.ds(0, gcols)], a_e, sin).wait()

        def fire_out(i, l_w, l_e, sout):
            off = pl.multiple_of((wid + i * _NW) * gwords, gwords)
            pltpu.async_copy(l_w, wo_hbm.at[pl.ds(off, gwords)], sout)
            pltpu.async_copy(l_e, eo_hbm.at[pl.ds(off, gwords)], sout)

        def wait_out(l_w, l_e, sout):
            pltpu.make_async_copy(
                l_w, wo_hbm.at[pl.ds(0, gwords)], sout).wait()
            pltpu.make_async_copy(
                l_e, eo_hbm.at[pl.ds(0, gwords)], sout).wait()

        def do_group(src_v, dst_v):
            @plsc.parallel_loop(0, d, unroll=2)
            def _(dd):
                base_d = iota * d + dd
                for k in range(gcols // _L):
                    val = src_v[dd, pl.ds(k * _L, _L)]
                    plsc.store_scatter(dst_v, [base_d + k * _L * d], val)

        fire_in(0, a_w0, a_e0, sin0)
        fire_in(1, a_w1, a_e1, sin1)

        def sub(p, i, valid_next, a_w, a_e, l_w, l_e, sin, sout):
            g = wid + i * _NW

            @pl.when(g < groups)
            def _():
                wait_in(a_w, a_e, sin)

                @pl.when(p > 0)
                def _():
                    wait_out(l_w, l_e, sout)

                do_group(a_w, l_w)
                do_group(a_e, l_e)
                fire_out(i, l_w, l_e, sout)

                @pl.when(valid_next & (g + 2 * _NW < groups))
                def _():
                    fire_in(i + 2, a_w, a_e, sin)

        def ploop(p, carry):
            sub(p, 2 * p, True, a_w0, a_e0, l_w0, l_e0, sin0, sout0)
            sub(p, 2 * p + 1, True, a_w1, a_e1, l_w1, l_e1, sin1, sout1)
            return carry

        lax.fori_loop(0, (iters + 1) // 2, ploop, 0)
        n0 = (groups - wid + 2 * _NW - 1) // (2 * _NW)
        n1 = (groups - wid - _NW + 2 * _NW - 1) // (2 * _NW)

        @pl.when(n0 > 0)
        def _():
            wait_out(l_w0, l_e0, sout0)

        @pl.when(n1 > 0)
        def _():
            wait_out(l_w1, l_e1, sout1)

        if vpad:
            @pl.when(wid == _NW - 1)
            def _():
                nw = vpad * d
                offt = full_blocks * 128 * d
                pltpu.sync_copy(wtail_hbm, l_w0.at[pl.ds(0, nw)])
                pltpu.sync_copy(etail_hbm, l_e0.at[pl.ds(0, nw)])
                pltpu.sync_copy(l_w0.at[pl.ds(0, nw)],
                                wo_hbm.at[pl.ds(offt, nw)])
                pltpu.sync_copy(l_e0.at[pl.ds(0, nw)],
                                eo_hbm.at[pl.ds(offt, nw)])

    return tr_call


def _build_sc_call(b, f, v, d):
    n = b * f
    assert n % (_NW * _CHUNK) == 0
    per_w = n // (_NW * _CHUNK)
    k = _CHUNK // _GATHER
    blocks_per_f = b // _CHUNK

    mesh = plsc.VectorSubcoreMesh(
        core_axis_name="c", subcore_axis_name="s",
        num_cores=_NUM_CORES, num_subcores=_NUM_SUBCORES)

    @functools.partial(
        pl.kernel,
        out_type=jax.ShapeDtypeStruct((f * d * b,), jnp.float32),
        mesh=mesh,
        compiler_params=pltpu.CompilerParams(
            use_tc_tiling_on_sc=False, needs_layout_passes=False),
        scratch_types=[
            [pltpu.VMEM((k, _GATHER), jnp.int32)] * 2,
            [pltpu.VMEM((_CHUNK, d), jnp.float32)] * 4,
            [pltpu.VMEM((d * _CHUNK,), jnp.float32)] * 2,
            pltpu.VMEM((_L,), jnp.float32),
            [pltpu.SemaphoreType.DMA] * 4,
        ],
    )
    def sc_call(idx_hbm, w_hbm, e_hbm, sig_hbm, out_hbm,
                idx_bufs, ge_bufs, o_bufs, sig_v, sems):
        wid = lax.axis_index("s") * _NUM_CORES + lax.axis_index("c")
        idx0, idx1 = idx_bufs
        w_v0, e_v0, w_v1, e_v1 = ge_bufs
        o_v0, o_v1 = o_bufs
        sin0, sout0, sin1, sout1 = sems
        pltpu.sync_copy(sig_hbm, sig_v)
        sig = sig_v[...]
        iota = lax.iota(jnp.int32, _L)

        def fire_in(i, idx_b, w_b, e_b, sin):
            cg = wid * per_w + i
            pltpu.sync_copy(
                idx_hbm.at[pl.ds(pl.multiple_of(cg * k, 8), k)], idx_b)
            for j in range(k):
                pltpu.async_copy(
                    w_hbm.at[idx_b.at[j]],
                    w_b.at[pl.ds(j * _GATHER, _GATHER)], sin)
                pltpu.async_copy(
                    e_hbm.at[idx_b.at[j]],
                    e_b.at[pl.ds(j * _GATHER, _GATHER)], sin)

        def wait_in(idx_b, w_b, e_b, sin):
            for j in range(k):
                pltpu.make_async_copy(
                    w_hbm.at[idx_b.at[j]],
                    w_b.at[pl.ds(j * _GATHER, _GATHER)], sin).wait()
                pltpu.make_async_copy(
                    e_hbm.at[idx_b.at[j]],
                    e_b.at[pl.ds(j * _GATHER, _GATHER)], sin).wait()

        ltile = (iota // 8) * (8 * _CHUNK) + (iota % 8) * _GATHER
        half = 8 * _CHUNK

        def fire_out(i, o_b, sout):
            cg = wid * per_w + i
            fi = cg // blocks_per_f
            tj0 = (cg % blocks_per_f) * (_CHUNK // _GATHER)
            for t in range(2):
                off = pl.multiple_of(
                    ((fi * 2 + t) * _GATHER + tj0) * (8 * _GATHER), 1024)
                pltpu.async_copy(o_b.at[pl.ds(t * half, half)],
                                 out_hbm.at[pl.ds(off, half)], sout)

        def wait_out(o_b, sout):
            for t in range(2):
                pltpu.make_async_copy(
                    o_b.at[pl.ds(t * half, half)],
                    out_hbm.at[pl.ds(0, half)], sout).wait()

        def compute(w_b, e_b, o_b):
            for jb in range(_CHUNK // _GATHER):
                @plsc.parallel_loop(0, _GATHER, unroll=8)
                def _(j, jb=jb):
                    row = jb * _GATHER + j
                    val = w_b[row] + sig * e_b[row]
                    plsc.store_scatter(o_b, [ltile + (jb * _CHUNK + j)],
                                       val)

        fire_in(0, idx0, w_v0, e_v0, sin0)
        fire_in(1, idx1, w_v1, e_v1, sin1)

        def ploop(p, carry):
            i = 2 * p
            wait_in(idx0, w_v0, e_v0, sin0)

            @pl.when(p > 0)
            def _():
                wait_out(o_v0, sout0)

            compute(w_v0, e_v0, o_v0)
            fire_out(i, o_v0, sout0)
            fire_in(i + 2, idx0, w_v0, e_v0, sin0)

            wait_in(idx1, w_v1, e_v1, sin1)

            @pl.when(p > 0)
            def _():
                wait_out(o_v1, sout1)

            compute(w_v1, e_v1, o_v1)
            fire_out(i + 1, o_v1, sout1)

            @pl.when(p < per_w // 2 - 1)
            def _():
                fire_in(i + 3, idx1, w_v1, e_v1, sin1)
            return carry

        lax.fori_loop(0, per_w // 2, ploop, 0)
        wait_in(idx0, w_v0, e_v0, sin0)
        wait_out(o_v0, sout0)
        compute(w_v0, e_v0, o_v0)
        fire_out(per_w - 1, o_v0, sout0)
        wait_out(o_v0, sout0)
        wait_out(o_v1, sout1)

    return sc_call


def kernel(x, weight, raw_sigma, eps):
    b, f = x.shape
    v, d = weight.shape
    n = b * f
    sigma = jax.nn.softplus(raw_sigma)
    sig_arr = jnp.full((_L,), sigma, dtype=jnp.float32)
    idx = x.T.reshape(n // _GATHER, _GATHER).astype(jnp.int32)
    tr_call = _build_transpose_call(v, d)
    vfull = (v // 128) * 128
    w1d, e1d = tr_call(weight.T, eps.T,
                       weight[vfull:, :].reshape(-1),
                       eps[vfull:, :].reshape(-1))
    w_lin = w1d.reshape(v, d)
    e_lin = e1d.reshape(v, d)
    sc_call = _build_sc_call(b, f, v, d)
    out1d = sc_call(idx, w_lin, e_lin, sig_arr)
    o5 = out1d.reshape(f, d // 8, b // _GATHER, 8, _GATHER)
    return jnp.transpose(o5, (2, 4, 0, 1, 3)).reshape(b, f, d)
